# Initial kernel scaffold; baseline (speedup 1.0000x reference)
import jax
import jax.numpy as jnp
from jax.experimental import pallas as pl


def kernel(hidden_states, gate_w, w_gate, w_up, w_down):
    raise NotImplementedError("write your pallas kernel here")


# speedup vs baseline: 1.2205x; 1.2205x over previous
import functools

import jax
import jax.numpy as jnp
from jax.experimental import pallas as pl
from jax.experimental.pallas import tpu as pltpu

T = 2048
D = 768
E = 8
FF = 1024
TOP_K = 2
T_TILE = 256
NT = T // T_TILE


def _moe_body(x_ref, gw_ref, wg_ref, wu_ref, wd_ref, out_ref, comb_ref):
    e = pl.program_id(0)
    t = pl.program_id(1)

    x = x_ref[...]

    @pl.when(jnp.logical_and(e == 0, t == 0))
    def _init():
        out_ref[...] = jnp.zeros_like(out_ref)

    @pl.when(e == 0)
    def _router():
        logits = jax.lax.dot_general(
            x, gw_ref[...], (((1,), (1,)), ((), ())),
            preferred_element_type=jnp.float32)
        probs = jax.nn.softmax(logits, axis=-1)
        eidx = jax.lax.broadcasted_iota(jnp.int32, (T_TILE, E), 1)
        m1 = jnp.max(probs, axis=-1, keepdims=True)
        i1 = jnp.min(jnp.where(probs == m1, eidx, E), axis=-1, keepdims=True)
        mask1 = eidx == i1
        p2 = jnp.where(mask1, -1.0, probs)
        m2 = jnp.max(p2, axis=-1, keepdims=True)
        i2 = jnp.min(jnp.where(p2 == m2, eidx, E), axis=-1, keepdims=True)
        mask2 = eidx == i2
        denom = m1 + m2
        comb = (jnp.where(mask1, m1, 0.0) + jnp.where(mask2, m2, 0.0)) / denom
        comb_ref[pl.ds(t * T_TILE, T_TILE), :] = comb

    g = jax.lax.dot_general(
        x, wg_ref[0], (((1,), (1,)), ((), ())),
        preferred_element_type=jnp.float32)
    u = jax.lax.dot_general(
        x, wu_ref[0], (((1,), (1,)), ((), ())),
        preferred_element_type=jnp.float32)
    act = g * jax.nn.sigmoid(g) * u
    o = jax.lax.dot_general(
        act, wd_ref[0], (((1,), (1,)), ((), ())),
        preferred_element_type=jnp.float32)
    comb = comb_ref[pl.ds(t * T_TILE, T_TILE), :]
    ecol = jax.lax.broadcasted_iota(jnp.int32, (T_TILE, E), 1)
    w = jnp.sum(jnp.where(ecol == e, comb, 0.0), axis=-1, keepdims=True)
    out_ref[pl.ds(t * T_TILE, T_TILE), :] += w * o


@functools.partial(jax.jit, static_argnames=("interpret",))
def kernel(hidden_states, gate_w, w_gate, w_up, w_down, interpret=False):
    grid = (E, NT)
    return pl.pallas_call(
        _moe_body,
        grid=grid,
        in_specs=[
            pl.BlockSpec((T_TILE, D), lambda e, t: (t, 0)),
            pl.BlockSpec((E, D), lambda e, t: (0, 0)),
            pl.BlockSpec((1, FF, D), lambda e, t: (e, 0, 0)),
            pl.BlockSpec((1, FF, D), lambda e, t: (e, 0, 0)),
            pl.BlockSpec((1, D, FF), lambda e, t: (e, 0, 0)),
        ],
        out_specs=pl.BlockSpec((T, D), lambda e, t: (0, 0)),
        out_shape=jax.ShapeDtypeStruct((T, D), jnp.float32),
        scratch_shapes=[pltpu.VMEM((T, E), jnp.float32)],
        interpret=interpret,
    )(hidden_states, gate_w, w_gate, w_up, w_down)
